# Initial kernel scaffold; baseline (speedup 1.0000x reference)
import jax
import jax.numpy as jnp
from jax.experimental import pallas as pl


def kernel(preds, gts):
    raise NotImplementedError("write your pallas kernel here")


# speedup vs baseline: 1.0527x; 1.0527x over previous
import jax
import jax.numpy as jnp
from jax import lax
from jax.experimental import pallas as pl
from jax.experimental.pallas import tpu as pltpu
from jax.experimental.pallas import tpu_sc as plsc

N = 21
B, H, W = 8, 512, 512
BH = 32
NBINS = N * N
PADB = 512
NC, NS = 2, 16
NW = NC * NS
VLEN = 16
TOTAL = B * H * W
CHUNK = TOTAL // NW


def _argmax_body(p_ref, g_ref, o_ref):
    x = p_ref[0]
    m = jnp.max(x, axis=0)
    ch = lax.broadcasted_iota(jnp.int32, (N, BH, W), 0)
    am = jnp.min(jnp.where(x == m[None], ch, N), axis=0)
    o_ref[0] = g_ref[0] * N + am


def _hist_body(idx_hbm, out_hbm, idx_v, hist_v, acc_v):
    wid = lax.axis_index("c") * NS + lax.axis_index("s")

    def zero(i, c):
        hist_v[pl.ds(i * VLEN, VLEN)] = jnp.zeros((VLEN,), jnp.float32)
        return c
    lax.fori_loop(0, (VLEN * PADB) // VLEN, zero, 0)

    pltpu.sync_copy(idx_hbm.at[pl.ds(wid * CHUNK, CHUNK)], idx_v)

    lanebase = lax.iota(jnp.int32, VLEN) * PADB
    ones = jnp.ones((VLEN,), jnp.float32)

    def body(i, c):
        v = idx_v[pl.ds(i * VLEN, VLEN)]
        plsc.addupdate_scatter(hist_v, [lanebase + v], ones)
        return c
    lax.fori_loop(0, CHUNK // VLEN, body, 0)

    def red(cc, c):
        def lanesum(l, a):
            return a + hist_v[pl.ds(l * PADB + cc * VLEN, VLEN)]
        acc_v[pl.ds(cc * VLEN, VLEN)] = lax.fori_loop(
            0, VLEN, lanesum, jnp.zeros((VLEN,), jnp.float32))
        return c
    lax.fori_loop(0, PADB // VLEN, red, 0)

    pltpu.sync_copy(acc_v, out_hbm.at[wid])


def _iou_body(p_ref, o_ref):
    conf = jnp.sum(p_ref[...], axis=0, keepdims=True)
    confb = jnp.broadcast_to(conf, (N, PADB))
    b = lax.broadcasted_iota(jnp.int32, (N, PADB), 1)
    c = lax.broadcasted_iota(jnp.int32, (N, PADB), 0)
    valid = b < NBINS
    rowm = ((b // N) == c) & valid
    colm = (lax.rem(b, N) == c) & valid
    tpm = rowm & colm
    z = jnp.zeros((N, PADB), jnp.float32)
    rowsum = jnp.sum(jnp.where(rowm, confb, z), axis=1, keepdims=True)
    colsum = jnp.sum(jnp.where(colm, confb, z), axis=1, keepdims=True)
    tp = jnp.sum(jnp.where(tpm, confb, z), axis=1, keepdims=True)
    denom = rowsum + colsum - tp
    present = denom > 0.0
    iou = jnp.where(present, tp / jnp.where(present, denom, 1.0), 0.0)
    miou = jnp.sum(iou) / jnp.sum(present.astype(jnp.float32))
    o_ref[0, 0] = miou


def _stage1(preds, gts):
    return pl.pallas_call(
        _argmax_body,
        grid=(B, H // BH),
        in_specs=[
            pl.BlockSpec((1, N, BH, W), lambda b, h: (b, 0, h, 0)),
            pl.BlockSpec((1, BH, W), lambda b, h: (b, h, 0)),
        ],
        out_specs=pl.BlockSpec((1, BH, W), lambda b, h: (b, h, 0)),
        out_shape=jax.ShapeDtypeStruct((B, H, W), jnp.int32),
    )(preds, gts)


def _stage2(idx_flat):
    mesh = plsc.VectorSubcoreMesh(core_axis_name="c", subcore_axis_name="s")
    return pl.kernel(
        _hist_body,
        out_type=jax.ShapeDtypeStruct((NW, PADB), jnp.float32),
        mesh=mesh,
        scratch_types=[
            pltpu.VMEM((CHUNK,), jnp.int32),
            pltpu.VMEM((VLEN * PADB,), jnp.float32),
            pltpu.VMEM((PADB,), jnp.float32),
        ],
        compiler_params=pltpu.CompilerParams(needs_layout_passes=False),
    )(idx_flat)


def _stage3(partials):
    return pl.pallas_call(
        _iou_body,
        out_shape=jax.ShapeDtypeStruct((1, 1), jnp.float32),
        out_specs=pl.BlockSpec(memory_space=pltpu.SMEM),
    )(partials)


def kernel(preds, gts):
    idx = _stage1(preds, gts.astype(jnp.int32))
    partials = _stage2(idx.reshape(-1))
    res = _stage3(partials)
    return res[0, 0]


# speedup vs baseline: 1.2366x; 1.1746x over previous
import jax
import jax.numpy as jnp
from jax import lax
from jax.experimental import pallas as pl
from jax.experimental.pallas import tpu as pltpu
from jax.experimental.pallas import tpu_sc as plsc

N = 21
B, H, W = 8, 512, 512
BH = 32
NBINS = N * N
PADB = 512
NC, NS = 2, 16
NW = NC * NS
VLEN = 16
TOTAL = B * H * W
CHUNK = TOTAL // NW


def _argmax_body(p_ref, g_ref, o_ref):
    x = p_ref[0]
    m = jnp.max(x, axis=0)
    ch = lax.broadcasted_iota(jnp.int32, (N, BH, W), 0)
    am = jnp.min(jnp.where(x == m[None], ch, N), axis=0)
    o_ref[0] = g_ref[0] * N + am


def _hist_body(idx_hbm, out_hbm, idx_v, hist_v, acc_v):
    wid = lax.axis_index("c") * NS + lax.axis_index("s")

    @plsc.parallel_loop(0, VLEN * PADB, step=VLEN, unroll=8)
    def _zero(i):
        hist_v[pl.ds(i, VLEN)] = jnp.zeros((VLEN,), jnp.float32)

    pltpu.sync_copy(idx_hbm.at[pl.ds(wid * CHUNK, CHUNK)], idx_v)

    lanebase = lax.iota(jnp.int32, VLEN) * PADB
    ones = jnp.ones((VLEN,), jnp.float32)

    @plsc.parallel_loop(0, CHUNK, step=VLEN, unroll=8)
    def _scat(i):
        v = idx_v[pl.ds(i, VLEN)]
        plsc.addupdate_scatter(hist_v, [lanebase + v], ones)

    @plsc.parallel_loop(0, PADB, step=VLEN, unroll=2)
    def _red(cc):
        acc = hist_v[pl.ds(cc, VLEN)]
        for l in range(1, VLEN):
            acc = acc + hist_v[pl.ds(l * PADB + cc, VLEN)]
        acc_v[pl.ds(cc, VLEN)] = acc

    pltpu.sync_copy(acc_v, out_hbm.at[wid])


def _iou_body(p_ref, o_ref):
    conf = jnp.sum(p_ref[...], axis=0, keepdims=True)
    confb = jnp.broadcast_to(conf, (N, PADB))
    b = lax.broadcasted_iota(jnp.int32, (N, PADB), 1)
    c = lax.broadcasted_iota(jnp.int32, (N, PADB), 0)
    valid = b < NBINS
    rowm = ((b // N) == c) & valid
    colm = (lax.rem(b, N) == c) & valid
    tpm = rowm & colm
    z = jnp.zeros((N, PADB), jnp.float32)
    rowsum = jnp.sum(jnp.where(rowm, confb, z), axis=1, keepdims=True)
    colsum = jnp.sum(jnp.where(colm, confb, z), axis=1, keepdims=True)
    tp = jnp.sum(jnp.where(tpm, confb, z), axis=1, keepdims=True)
    denom = rowsum + colsum - tp
    present = denom > 0.0
    iou = jnp.where(present, tp / jnp.where(present, denom, 1.0), 0.0)
    miou = jnp.sum(iou) / jnp.sum(present.astype(jnp.float32))
    o_ref[0, 0] = miou


def _stage1(preds, gts):
    return pl.pallas_call(
        _argmax_body,
        grid=(B, H // BH),
        in_specs=[
            pl.BlockSpec((1, N, BH, W), lambda b, h: (b, 0, h, 0)),
            pl.BlockSpec((1, BH, W), lambda b, h: (b, h, 0)),
        ],
        out_specs=pl.BlockSpec((1, BH, W), lambda b, h: (b, h, 0)),
        out_shape=jax.ShapeDtypeStruct((B, H, W), jnp.int32),
    )(preds, gts)


def _stage2(idx_flat):
    mesh = plsc.VectorSubcoreMesh(core_axis_name="c", subcore_axis_name="s")
    return pl.kernel(
        _hist_body,
        out_type=jax.ShapeDtypeStruct((NW, PADB), jnp.float32),
        mesh=mesh,
        scratch_types=[
            pltpu.VMEM((CHUNK,), jnp.int32),
            pltpu.VMEM((VLEN * PADB,), jnp.float32),
            pltpu.VMEM((PADB,), jnp.float32),
        ],
        compiler_params=pltpu.CompilerParams(needs_layout_passes=False),
    )(idx_flat)


def _stage3(partials):
    return pl.pallas_call(
        _iou_body,
        out_shape=jax.ShapeDtypeStruct((1, 1), jnp.float32),
        out_specs=pl.BlockSpec(memory_space=pltpu.SMEM),
    )(partials)


def kernel(preds, gts):
    idx = _stage1(preds, gts.astype(jnp.int32))
    partials = _stage2(idx.reshape(-1))
    res = _stage3(partials)
    return res[0, 0]


# speedup vs baseline: 1.2382x; 1.0013x over previous
import jax
import jax.numpy as jnp
from jax import lax
from jax.experimental import pallas as pl
from jax.experimental.pallas import tpu as pltpu
from jax.experimental.pallas import tpu_sc as plsc

N = 21
B, H, W = 8, 512, 512
BH = 32
NBINS = N * N
PADB = 512
NC, NS = 2, 16
NW = NC * NS
VLEN = 16
TOTAL = B * H * W
CHUNK = TOTAL // NW


def _argmax_body(p_ref, g_ref, o_ref):
    x = p_ref[0]
    m = jnp.max(x, axis=0)
    ch = lax.broadcasted_iota(jnp.int32, (N, BH, W), 0)
    am = jnp.min(jnp.where(x == m[None], ch, N), axis=0)
    o_ref[0] = g_ref[0] * N + am


def _hist_body(idx_hbm, out_hbm, idx_v, hist_v, acc_v):
    wid = lax.axis_index("c") * NS + lax.axis_index("s")

    @plsc.parallel_loop(0, VLEN * PADB, step=VLEN, unroll=8)
    def _zero(i):
        hist_v[pl.ds(i, VLEN)] = jnp.zeros((VLEN,), jnp.float32)

    pltpu.sync_copy(idx_hbm.at[pl.ds(wid * CHUNK, CHUNK)], idx_v)

    lanebase = lax.iota(jnp.int32, VLEN) * PADB
    ones = jnp.ones((VLEN,), jnp.float32)

    @plsc.parallel_loop(0, CHUNK, step=VLEN, unroll=8)
    def _scat(i):
        v = idx_v[pl.ds(i, VLEN)]
        plsc.addupdate_scatter(hist_v, [lanebase + v], ones)

    @plsc.parallel_loop(0, PADB, step=VLEN, unroll=2)
    def _red(cc):
        acc = hist_v[pl.ds(cc, VLEN)]
        for l in range(1, VLEN):
            acc = acc + hist_v[pl.ds(l * PADB + cc, VLEN)]
        acc_v[pl.ds(cc, VLEN)] = acc

    pltpu.sync_copy(acc_v, out_hbm.at[wid])


def _iou_body(p_ref, o_ref):
    conf = jnp.sum(p_ref[...], axis=0, keepdims=True)
    confb = jnp.broadcast_to(conf, (N, PADB))
    b = lax.broadcasted_iota(jnp.int32, (N, PADB), 1)
    c = lax.broadcasted_iota(jnp.int32, (N, PADB), 0)
    valid = b < NBINS
    rowm = ((b // N) == c) & valid
    colm = (lax.rem(b, N) == c) & valid
    tpm = rowm & colm
    z = jnp.zeros((N, PADB), jnp.float32)
    rowsum = jnp.sum(jnp.where(rowm, confb, z), axis=1, keepdims=True)
    colsum = jnp.sum(jnp.where(colm, confb, z), axis=1, keepdims=True)
    tp = jnp.sum(jnp.where(tpm, confb, z), axis=1, keepdims=True)
    denom = rowsum + colsum - tp
    present = denom > 0.0
    iou = jnp.where(present, tp / jnp.where(present, denom, 1.0), 0.0)
    miou = jnp.sum(iou) / jnp.sum(present.astype(jnp.float32))
    o_ref[0, 0] = miou


def _stage1(preds, gts):
    return pl.pallas_call(
        _argmax_body,
        grid=(B, H // BH),
        in_specs=[
            pl.BlockSpec((1, N, BH, W), lambda b, h: (b, 0, h, 0)),
            pl.BlockSpec((1, BH, W), lambda b, h: (b, h, 0)),
        ],
        out_specs=pl.BlockSpec((1, BH, W), lambda b, h: (b, h, 0)),
        out_shape=jax.ShapeDtypeStruct((B, H, W), jnp.int32),
    )(preds, gts)


def _stage2(idx_flat):
    mesh = plsc.VectorSubcoreMesh(core_axis_name="c", subcore_axis_name="s")
    return pl.kernel(
        _hist_body,
        out_type=jax.ShapeDtypeStruct((NW, PADB), jnp.float32),
        mesh=mesh,
        scratch_types=[
            pltpu.VMEM((CHUNK,), jnp.int32),
            pltpu.VMEM((VLEN * PADB,), jnp.float32),
            pltpu.VMEM((PADB,), jnp.float32),
        ],
        compiler_params=pltpu.CompilerParams(
            needs_layout_passes=False, use_tc_tiling_on_sc=True),
    )(idx_flat)


def _stage3(partials):
    return pl.pallas_call(
        _iou_body,
        out_shape=jax.ShapeDtypeStruct((1, 1), jnp.float32),
        out_specs=pl.BlockSpec(memory_space=pltpu.SMEM),
    )(partials)


def kernel(preds, gts):
    idx = _stage1(preds, gts.astype(jnp.int32))
    partials = _stage2(idx.reshape(-1))
    res = _stage3(partials)
    return res[0, 0]


# speedup vs baseline: 1.3276x; 1.0722x over previous
import jax
import jax.numpy as jnp
from jax import lax
from jax.experimental import pallas as pl
from jax.experimental.pallas import tpu as pltpu
from jax.experimental.pallas import tpu_sc as plsc

N = 21
B, H, W = 8, 512, 512
BH = 32
NBINS = N * N
PADB = 512
NC, NS = 2, 16
NW = NC * NS
VLEN = 16
TOTAL = B * H * W
CHUNK = TOTAL // NW


def _argmax_body(p_ref, g_ref, o_ref):
    x = p_ref[0]
    m = jnp.max(x, axis=0)
    ch = lax.broadcasted_iota(jnp.int32, (N, BH, W), 0)
    am = jnp.min(jnp.where(x == m[None], ch, N), axis=0)
    o_ref[0] = g_ref[0] * N + am


def _hist_body(idx_hbm, out_hbm, idx_v, hist_v, acc_v):
    wid = lax.axis_index("c") * NS + lax.axis_index("s")

    @plsc.parallel_loop(0, VLEN * PADB, step=VLEN, unroll=8)
    def _zero(i):
        hist_v[pl.ds(i, VLEN)] = jnp.zeros((VLEN,), jnp.float32)

    b = wid // 4
    r0 = (wid % 4) * 128
    pltpu.sync_copy(idx_hbm.at[b, pl.ds(r0, 128), :], idx_v)

    lanebase = lax.iota(jnp.int32, VLEN) * PADB
    ones = jnp.ones((VLEN,), jnp.float32)

    @plsc.parallel_loop(0, 128, step=1, unroll=2)
    def _scat(i):
        for c in range(W // VLEN):
            v = idx_v[i, pl.ds(c * VLEN, VLEN)]
            plsc.addupdate_scatter(hist_v, [lanebase + v], ones)

    @plsc.parallel_loop(0, PADB, step=VLEN, unroll=2)
    def _red(cc):
        acc = hist_v[pl.ds(cc, VLEN)]
        for l in range(1, VLEN):
            acc = acc + hist_v[pl.ds(l * PADB + cc, VLEN)]
        acc_v[pl.ds(cc, VLEN)] = acc

    pltpu.sync_copy(acc_v, out_hbm.at[wid])


def _iou_body(p_ref, o_ref):
    conf = jnp.sum(p_ref[...], axis=0, keepdims=True)
    confb = jnp.broadcast_to(conf, (N, PADB))
    b = lax.broadcasted_iota(jnp.int32, (N, PADB), 1)
    c = lax.broadcasted_iota(jnp.int32, (N, PADB), 0)
    valid = b < NBINS
    rowm = ((b // N) == c) & valid
    colm = (lax.rem(b, N) == c) & valid
    tpm = rowm & colm
    z = jnp.zeros((N, PADB), jnp.float32)
    rowsum = jnp.sum(jnp.where(rowm, confb, z), axis=1, keepdims=True)
    colsum = jnp.sum(jnp.where(colm, confb, z), axis=1, keepdims=True)
    tp = jnp.sum(jnp.where(tpm, confb, z), axis=1, keepdims=True)
    denom = rowsum + colsum - tp
    present = denom > 0.0
    iou = jnp.where(present, tp / jnp.where(present, denom, 1.0), 0.0)
    miou = jnp.sum(iou) / jnp.sum(present.astype(jnp.float32))
    o_ref[0, 0] = miou


def _stage1(preds, gts):
    return pl.pallas_call(
        _argmax_body,
        grid=(B, H // BH),
        in_specs=[
            pl.BlockSpec((1, N, BH, W), lambda b, h: (b, 0, h, 0)),
            pl.BlockSpec((1, BH, W), lambda b, h: (b, h, 0)),
        ],
        out_specs=pl.BlockSpec((1, BH, W), lambda b, h: (b, h, 0)),
        out_shape=jax.ShapeDtypeStruct((B, H, W), jnp.int32),
    )(preds, gts)


def _stage2(idx_flat):
    mesh = plsc.VectorSubcoreMesh(core_axis_name="c", subcore_axis_name="s")
    return pl.kernel(
        _hist_body,
        out_type=jax.ShapeDtypeStruct((NW, PADB), jnp.float32),
        mesh=mesh,
        scratch_types=[
            pltpu.VMEM((128, W), jnp.int32),
            pltpu.VMEM((VLEN * PADB,), jnp.float32),
            pltpu.VMEM((PADB,), jnp.float32),
        ],
        compiler_params=pltpu.CompilerParams(
            needs_layout_passes=False, use_tc_tiling_on_sc=True),
    )(idx_flat)


def _stage3(partials):
    return pl.pallas_call(
        _iou_body,
        out_shape=jax.ShapeDtypeStruct((1, 1), jnp.float32),
        out_specs=pl.BlockSpec(memory_space=pltpu.SMEM),
    )(partials)


def kernel(preds, gts):
    idx = _stage1(preds, gts.astype(jnp.int32))
    partials = _stage2(idx)
    res = _stage3(partials)
    return res[0, 0]


# speedup vs baseline: 1.3500x; 1.0169x over previous
import jax
import jax.numpy as jnp
from jax import lax
from jax.experimental import pallas as pl
from jax.experimental.pallas import tpu as pltpu
from jax.experimental.pallas import tpu_sc as plsc

N = 21
B, H, W = 8, 512, 512
BH = 32
NBINS = N * N
PADB = 512
NC, NS = 2, 16
NW = NC * NS
VLEN = 16
TOTAL = B * H * W
CHUNK = TOTAL // NW


def _argmax_body(p_ref, g_ref, o_ref):
    x = p_ref[0]
    m = jnp.max(x, axis=0)
    ch = lax.broadcasted_iota(jnp.int32, (N, BH, W), 0)
    am = jnp.min(jnp.where(x == m[None], ch, N), axis=0)
    o_ref[0] = g_ref[0] * N + am


def _hist_body(idx_hbm, out_hbm, idx_v, hist_v, acc_v):
    nb = idx_hbm.shape[0]
    slabs = NW // nb
    rows = H // slabs
    wid = lax.axis_index("c") * NS + lax.axis_index("s")

    @plsc.parallel_loop(0, VLEN * PADB, step=VLEN, unroll=8)
    def _zero(i):
        hist_v[pl.ds(i, VLEN)] = jnp.zeros((VLEN,), jnp.float32)

    b = wid // slabs
    r0 = (wid % slabs) * rows
    pltpu.sync_copy(idx_hbm.at[b, pl.ds(r0, rows), :], idx_v)

    lanebase = lax.iota(jnp.int32, VLEN) * PADB
    ones = jnp.ones((VLEN,), jnp.float32)

    @plsc.parallel_loop(0, rows, step=1, unroll=2)
    def _scat(i):
        for c in range(W // VLEN):
            v = idx_v[i, pl.ds(c * VLEN, VLEN)]
            plsc.addupdate_scatter(hist_v, [lanebase + v], ones)

    @plsc.parallel_loop(0, PADB, step=VLEN, unroll=2)
    def _red(cc):
        acc = hist_v[pl.ds(cc, VLEN)]
        for l in range(1, VLEN):
            acc = acc + hist_v[pl.ds(l * PADB + cc, VLEN)]
        acc_v[pl.ds(cc, VLEN)] = acc

    pltpu.sync_copy(acc_v, out_hbm.at[wid])


def _iou_from_conf(conf, o_ref):
    confb = jnp.broadcast_to(conf, (N, PADB))
    b = lax.broadcasted_iota(jnp.int32, (N, PADB), 1)
    c = lax.broadcasted_iota(jnp.int32, (N, PADB), 0)
    valid = b < NBINS
    rowm = ((b // N) == c) & valid
    colm = (lax.rem(b, N) == c) & valid
    tpm = rowm & colm
    z = jnp.zeros((N, PADB), jnp.float32)
    rowsum = jnp.sum(jnp.where(rowm, confb, z), axis=1, keepdims=True)
    colsum = jnp.sum(jnp.where(colm, confb, z), axis=1, keepdims=True)
    tp = jnp.sum(jnp.where(tpm, confb, z), axis=1, keepdims=True)
    denom = rowsum + colsum - tp
    present = denom > 0.0
    iou = jnp.where(present, tp / jnp.where(present, denom, 1.0), 0.0)
    miou = jnp.sum(iou) / jnp.sum(present.astype(jnp.float32))
    o_ref[0, 0] = miou


def _stage1(preds, gts, b0, nb):
    return pl.pallas_call(
        _argmax_body,
        grid=(nb, H // BH),
        in_specs=[
            pl.BlockSpec((1, N, BH, W), lambda b, h: (b + b0, 0, h, 0)),
            pl.BlockSpec((1, BH, W), lambda b, h: (b + b0, h, 0)),
        ],
        out_specs=pl.BlockSpec((1, BH, W), lambda b, h: (b, h, 0)),
        out_shape=jax.ShapeDtypeStruct((nb, H, W), jnp.int32),
    )(preds, gts)


def _stage2(idx):
    nb = idx.shape[0]
    rows = H // (NW // nb)
    mesh = plsc.VectorSubcoreMesh(core_axis_name="c", subcore_axis_name="s")
    return pl.kernel(
        _hist_body,
        out_type=jax.ShapeDtypeStruct((NW, PADB), jnp.float32),
        mesh=mesh,
        scratch_types=[
            pltpu.VMEM((rows, W), jnp.int32),
            pltpu.VMEM((VLEN * PADB,), jnp.float32),
            pltpu.VMEM((PADB,), jnp.float32),
        ],
        compiler_params=pltpu.CompilerParams(
            needs_layout_passes=False, use_tc_tiling_on_sc=True),
    )(idx)


def _iou2_body(pa_ref, pb_ref, o_ref):
    _iou_from_conf(
        jnp.sum(pa_ref[...], axis=0, keepdims=True)
        + jnp.sum(pb_ref[...], axis=0, keepdims=True), o_ref)


def _stage3(pa, pb):
    return pl.pallas_call(
        _iou2_body,
        out_shape=jax.ShapeDtypeStruct((1, 1), jnp.float32),
        out_specs=pl.BlockSpec(memory_space=pltpu.SMEM),
    )(pa, pb)


def kernel(preds, gts):
    gi = gts.astype(jnp.int32)
    half = B // 2
    idx_a = _stage1(preds, gi, 0, half)
    pa = _stage2(idx_a)
    idx_b = _stage1(preds, gi, half, half)
    pb = _stage2(idx_b)
    res = _stage3(pa, pb)
    return res[0, 0]


# speedup vs baseline: 1.7509x; 1.2969x over previous
import jax
import jax.numpy as jnp
from jax import lax
from jax.experimental import pallas as pl
from jax.experimental.pallas import tpu as pltpu
from jax.experimental.pallas import tpu_sc as plsc

N = 21
B, H, W = 8, 512, 512
BH = 64
NBINS = N * N
PADB = 512
NC, NS = 2, 16
NW = NC * NS
VLEN = 16
TOTAL = B * H * W
CHUNK = TOTAL // NW


def _argmax_body(p_ref, g_ref, o_ref):
    x = p_ref[0]
    m = jnp.max(x, axis=0)
    ch = lax.broadcasted_iota(jnp.int32, (N, BH, W), 0)
    am = jnp.min(jnp.where(x == m[None], ch, N), axis=0)
    o_ref[0] = g_ref[0] * N + am


def _hist_body(idx_hbm, out_hbm, idx_v, hist_v, acc_v):
    nb = idx_hbm.shape[0]
    slabs = NW // nb
    rows = H // slabs
    wid = lax.axis_index("c") * NS + lax.axis_index("s")

    @plsc.parallel_loop(0, VLEN * PADB, step=VLEN, unroll=8)
    def _zero(i):
        hist_v[pl.ds(i, VLEN)] = jnp.zeros((VLEN,), jnp.float32)

    b = wid // slabs
    r0 = (wid % slabs) * rows
    pltpu.sync_copy(idx_hbm.at[b, pl.ds(r0, rows), :], idx_v)

    lanebase = lax.iota(jnp.int32, VLEN) * PADB
    ones = jnp.ones((VLEN,), jnp.float32)

    @plsc.parallel_loop(0, rows, step=1, unroll=2)
    def _scat(i):
        for c in range(W // VLEN):
            v = idx_v[i, pl.ds(c * VLEN, VLEN)]
            plsc.addupdate_scatter(hist_v, [lanebase + v], ones)

    @plsc.parallel_loop(0, PADB, step=VLEN, unroll=2)
    def _red(cc):
        acc = hist_v[pl.ds(cc, VLEN)]
        for l in range(1, VLEN):
            acc = acc + hist_v[pl.ds(l * PADB + cc, VLEN)]
        acc_v[pl.ds(cc, VLEN)] = acc

    pltpu.sync_copy(acc_v, out_hbm.at[wid])


def _iou_from_conf(conf, o_ref):
    confb = jnp.broadcast_to(conf, (N, PADB))
    b = lax.broadcasted_iota(jnp.int32, (N, PADB), 1)
    c = lax.broadcasted_iota(jnp.int32, (N, PADB), 0)
    valid = b < NBINS
    rowm = ((b // N) == c) & valid
    colm = (lax.rem(b, N) == c) & valid
    tpm = rowm & colm
    z = jnp.zeros((N, PADB), jnp.float32)
    rowsum = jnp.sum(jnp.where(rowm, confb, z), axis=1, keepdims=True)
    colsum = jnp.sum(jnp.where(colm, confb, z), axis=1, keepdims=True)
    tp = jnp.sum(jnp.where(tpm, confb, z), axis=1, keepdims=True)
    denom = rowsum + colsum - tp
    present = denom > 0.0
    iou = jnp.where(present, tp / jnp.where(present, denom, 1.0), 0.0)
    miou = jnp.sum(iou) / jnp.sum(present.astype(jnp.float32))
    o_ref[0, 0] = miou


def _stage1(preds, gts, b0, nb):
    return pl.pallas_call(
        _argmax_body,
        grid=(nb, H // BH),
        in_specs=[
            pl.BlockSpec((1, N, BH, W), lambda b, h: (b + b0, 0, h, 0)),
            pl.BlockSpec((1, BH, W), lambda b, h: (b + b0, h, 0)),
        ],
        out_specs=pl.BlockSpec((1, BH, W), lambda b, h: (b, h, 0)),
        out_shape=jax.ShapeDtypeStruct((nb, H, W), jnp.int32),
    )(preds, gts)


def _stage2(idx):
    nb = idx.shape[0]
    rows = H // (NW // nb)
    mesh = plsc.VectorSubcoreMesh(core_axis_name="c", subcore_axis_name="s")
    return pl.kernel(
        _hist_body,
        out_type=jax.ShapeDtypeStruct((NW, PADB), jnp.float32),
        mesh=mesh,
        scratch_types=[
            pltpu.VMEM((rows, W), jnp.int32),
            pltpu.VMEM((VLEN * PADB,), jnp.float32),
            pltpu.VMEM((PADB,), jnp.float32),
        ],
        compiler_params=pltpu.CompilerParams(
            needs_layout_passes=False, use_tc_tiling_on_sc=True),
    )(idx)


def _iou2_body(pa_ref, pb_ref, o_ref):
    _iou_from_conf(
        jnp.sum(pa_ref[...], axis=0, keepdims=True)
        + jnp.sum(pb_ref[...], axis=0, keepdims=True), o_ref)


def _stage3(pa, pb):
    return pl.pallas_call(
        _iou2_body,
        out_shape=jax.ShapeDtypeStruct((1, 1), jnp.float32),
        out_specs=pl.BlockSpec(memory_space=pltpu.SMEM),
    )(pa, pb)


def kernel(preds, gts):
    gi = gts.astype(jnp.int32)
    half = B // 2
    idx_a = _stage1(preds, gi, 0, half)
    pa = _stage2(idx_a)
    idx_b = _stage1(preds, gi, half, half)
    pb = _stage2(idx_b)
    res = _stage3(pa, pb)
    return res[0, 0]


# speedup vs baseline: 2.0546x; 1.1735x over previous
import jax
import jax.numpy as jnp
from jax import lax
from jax.experimental import pallas as pl
from jax.experimental.pallas import tpu as pltpu
from jax.experimental.pallas import tpu_sc as plsc

N = 21
B, H, W = 8, 512, 512
BH = 128
NBINS = N * N
PADB = 512
NC, NS = 2, 16
NW = NC * NS
VLEN = 16
TOTAL = B * H * W
CHUNK = TOTAL // NW


def _argmax_body(p_ref, g_ref, o_ref):
    x = p_ref[0]
    m = jnp.max(x, axis=0)
    ch = lax.broadcasted_iota(jnp.int32, (N, BH, W), 0)
    am = jnp.min(jnp.where(x == m[None], ch, N), axis=0)
    o_ref[0] = g_ref[0] * N + am


def _hist_body(idx_hbm, out_hbm, idx_v, hist_v, acc_v):
    nb = idx_hbm.shape[0]
    slabs = NW // nb
    rows = H // slabs
    wid = lax.axis_index("c") * NS + lax.axis_index("s")

    @plsc.parallel_loop(0, VLEN * PADB, step=VLEN, unroll=8)
    def _zero(i):
        hist_v[pl.ds(i, VLEN)] = jnp.zeros((VLEN,), jnp.float32)

    b = wid // slabs
    r0 = (wid % slabs) * rows
    pltpu.sync_copy(idx_hbm.at[b, pl.ds(r0, rows), :], idx_v)

    lanebase = lax.iota(jnp.int32, VLEN) * PADB
    ones = jnp.ones((VLEN,), jnp.float32)

    @plsc.parallel_loop(0, rows, step=1, unroll=2)
    def _scat(i):
        for c in range(W // VLEN):
            v = idx_v[i, pl.ds(c * VLEN, VLEN)]
            plsc.addupdate_scatter(hist_v, [lanebase + v], ones)

    @plsc.parallel_loop(0, PADB, step=VLEN, unroll=2)
    def _red(cc):
        acc = hist_v[pl.ds(cc, VLEN)]
        for l in range(1, VLEN):
            acc = acc + hist_v[pl.ds(l * PADB + cc, VLEN)]
        acc_v[pl.ds(cc, VLEN)] = acc

    pltpu.sync_copy(acc_v, out_hbm.at[wid])


def _iou_from_conf(conf, o_ref):
    confb = jnp.broadcast_to(conf, (N, PADB))
    b = lax.broadcasted_iota(jnp.int32, (N, PADB), 1)
    c = lax.broadcasted_iota(jnp.int32, (N, PADB), 0)
    valid = b < NBINS
    rowm = ((b // N) == c) & valid
    colm = (lax.rem(b, N) == c) & valid
    tpm = rowm & colm
    z = jnp.zeros((N, PADB), jnp.float32)
    rowsum = jnp.sum(jnp.where(rowm, confb, z), axis=1, keepdims=True)
    colsum = jnp.sum(jnp.where(colm, confb, z), axis=1, keepdims=True)
    tp = jnp.sum(jnp.where(tpm, confb, z), axis=1, keepdims=True)
    denom = rowsum + colsum - tp
    present = denom > 0.0
    iou = jnp.where(present, tp / jnp.where(present, denom, 1.0), 0.0)
    miou = jnp.sum(iou) / jnp.sum(present.astype(jnp.float32))
    o_ref[0, 0] = miou


def _stage1(preds, gts, b0, nb):
    return pl.pallas_call(
        _argmax_body,
        grid=(nb, H // BH),
        in_specs=[
            pl.BlockSpec((1, N, BH, W), lambda b, h: (b + b0, 0, h, 0)),
            pl.BlockSpec((1, BH, W), lambda b, h: (b + b0, h, 0)),
        ],
        out_specs=pl.BlockSpec((1, BH, W), lambda b, h: (b, h, 0)),
        out_shape=jax.ShapeDtypeStruct((nb, H, W), jnp.int32),
    )(preds, gts)


def _stage2(idx):
    nb = idx.shape[0]
    rows = H // (NW // nb)
    mesh = plsc.VectorSubcoreMesh(core_axis_name="c", subcore_axis_name="s")
    return pl.kernel(
        _hist_body,
        out_type=jax.ShapeDtypeStruct((NW, PADB), jnp.float32),
        mesh=mesh,
        scratch_types=[
            pltpu.VMEM((rows, W), jnp.int32),
            pltpu.VMEM((VLEN * PADB,), jnp.float32),
            pltpu.VMEM((PADB,), jnp.float32),
        ],
        compiler_params=pltpu.CompilerParams(
            needs_layout_passes=False, use_tc_tiling_on_sc=True),
    )(idx)


def _iou2_body(pa_ref, pb_ref, o_ref):
    _iou_from_conf(
        jnp.sum(pa_ref[...], axis=0, keepdims=True)
        + jnp.sum(pb_ref[...], axis=0, keepdims=True), o_ref)


def _stage3(pa, pb):
    return pl.pallas_call(
        _iou2_body,
        out_shape=jax.ShapeDtypeStruct((1, 1), jnp.float32),
        out_specs=pl.BlockSpec(memory_space=pltpu.SMEM),
    )(pa, pb)


def kernel(preds, gts):
    gi = gts.astype(jnp.int32)
    half = B // 2
    idx_a = _stage1(preds, gi, 0, half)
    pa = _stage2(idx_a)
    idx_b = _stage1(preds, gi, half, half)
    pb = _stage2(idx_b)
    res = _stage3(pa, pb)
    return res[0, 0]


# speedup vs baseline: 2.1001x; 1.0222x over previous
import jax
import jax.numpy as jnp
from jax import lax
from jax.experimental import pallas as pl
from jax.experimental.pallas import tpu as pltpu
from jax.experimental.pallas import tpu_sc as plsc

N = 21
B, H, W = 8, 512, 512
BH = 256
NBINS = N * N
PADB = 512
NC, NS = 2, 16
NW = NC * NS
VLEN = 16
TOTAL = B * H * W
CHUNK = TOTAL // NW


def _argmax_body(p_ref, g_ref, o_ref):
    x = p_ref[0]
    m = jnp.max(x, axis=0)
    ch = lax.broadcasted_iota(jnp.int32, (N, BH, W), 0)
    am = jnp.min(jnp.where(x == m[None], ch, N), axis=0)
    o_ref[0] = g_ref[0] * N + am


def _hist_body(idx_hbm, out_hbm, idx_v, hist_v, acc_v):
    nb = idx_hbm.shape[0]
    slabs = NW // nb
    rows = H // slabs
    wid = lax.axis_index("c") * NS + lax.axis_index("s")

    @plsc.parallel_loop(0, VLEN * PADB, step=VLEN, unroll=8)
    def _zero(i):
        hist_v[pl.ds(i, VLEN)] = jnp.zeros((VLEN,), jnp.float32)

    b = wid // slabs
    r0 = (wid % slabs) * rows
    pltpu.sync_copy(idx_hbm.at[b, pl.ds(r0, rows), :], idx_v)

    lanebase = lax.iota(jnp.int32, VLEN) * PADB
    ones = jnp.ones((VLEN,), jnp.float32)

    @plsc.parallel_loop(0, rows, step=1, unroll=2)
    def _scat(i):
        for c in range(W // VLEN):
            v = idx_v[i, pl.ds(c * VLEN, VLEN)]
            plsc.addupdate_scatter(hist_v, [lanebase + v], ones)

    @plsc.parallel_loop(0, PADB, step=VLEN, unroll=2)
    def _red(cc):
        acc = hist_v[pl.ds(cc, VLEN)]
        for l in range(1, VLEN):
            acc = acc + hist_v[pl.ds(l * PADB + cc, VLEN)]
        acc_v[pl.ds(cc, VLEN)] = acc

    pltpu.sync_copy(acc_v, out_hbm.at[wid])


def _iou_from_conf(conf, o_ref):
    confb = jnp.broadcast_to(conf, (N, PADB))
    b = lax.broadcasted_iota(jnp.int32, (N, PADB), 1)
    c = lax.broadcasted_iota(jnp.int32, (N, PADB), 0)
    valid = b < NBINS
    rowm = ((b // N) == c) & valid
    colm = (lax.rem(b, N) == c) & valid
    tpm = rowm & colm
    z = jnp.zeros((N, PADB), jnp.float32)
    rowsum = jnp.sum(jnp.where(rowm, confb, z), axis=1, keepdims=True)
    colsum = jnp.sum(jnp.where(colm, confb, z), axis=1, keepdims=True)
    tp = jnp.sum(jnp.where(tpm, confb, z), axis=1, keepdims=True)
    denom = rowsum + colsum - tp
    present = denom > 0.0
    iou = jnp.where(present, tp / jnp.where(present, denom, 1.0), 0.0)
    miou = jnp.sum(iou) / jnp.sum(present.astype(jnp.float32))
    o_ref[0, 0] = miou


def _stage1(preds, gts, b0, nb):
    return pl.pallas_call(
        _argmax_body,
        grid=(nb, H // BH),
        in_specs=[
            pl.BlockSpec((1, N, BH, W), lambda b, h: (b + b0, 0, h, 0)),
            pl.BlockSpec((1, BH, W), lambda b, h: (b + b0, h, 0)),
        ],
        out_specs=pl.BlockSpec((1, BH, W), lambda b, h: (b, h, 0)),
        out_shape=jax.ShapeDtypeStruct((nb, H, W), jnp.int32),
    )(preds, gts)


def _stage2(idx):
    nb = idx.shape[0]
    rows = H // (NW // nb)
    mesh = plsc.VectorSubcoreMesh(core_axis_name="c", subcore_axis_name="s")
    return pl.kernel(
        _hist_body,
        out_type=jax.ShapeDtypeStruct((NW, PADB), jnp.float32),
        mesh=mesh,
        scratch_types=[
            pltpu.VMEM((rows, W), jnp.int32),
            pltpu.VMEM((VLEN * PADB,), jnp.float32),
            pltpu.VMEM((PADB,), jnp.float32),
        ],
        compiler_params=pltpu.CompilerParams(
            needs_layout_passes=False, use_tc_tiling_on_sc=True),
    )(idx)


def _iou2_body(pa_ref, pb_ref, o_ref):
    _iou_from_conf(
        jnp.sum(pa_ref[...], axis=0, keepdims=True)
        + jnp.sum(pb_ref[...], axis=0, keepdims=True), o_ref)


def _stage3(pa, pb):
    return pl.pallas_call(
        _iou2_body,
        out_shape=jax.ShapeDtypeStruct((1, 1), jnp.float32),
        out_specs=pl.BlockSpec(memory_space=pltpu.SMEM),
    )(pa, pb)


def kernel(preds, gts):
    gi = gts.astype(jnp.int32)
    half = B // 2
    idx_a = _stage1(preds, gi, 0, half)
    pa = _stage2(idx_a)
    idx_b = _stage1(preds, gi, half, half)
    pb = _stage2(idx_b)
    res = _stage3(pa, pb)
    return res[0, 0]
